# baseline (speedup 1.0000x reference)
import functools

import jax
import jax.numpy as jnp
from jax import lax
from jax.experimental import pallas as pl
from jax.experimental.pallas import tpu as pltpu
from jax.experimental.pallas import tpu_sc as plsc

N = 10000
E = 640000
NC = 2
NS = 16
NW = NC * NS
EPW = E // NW
K = 80
ITERS = EPW // K
NROWS = 1000
NSPLIT = N // NROWS

_mesh = plsc.VectorSubcoreMesh(core_axis_name="c", subcore_axis_name="s")


def _deg_body(dst_hbm, ew_hbm, pdeg0_hbm, pdeg1_hbm, idx_v, val_v, zbuf,
              deg_sh):
    c = lax.axis_index("c")
    s = lax.axis_index("s")
    w = c * NS + s

    def zfill(i, cy):
        zbuf[pl.ds(i * 16, 16)] = jnp.zeros((16,), jnp.float32)
        return cy

    lax.fori_loop(0, 64, zfill, 0)

    @pl.when(s < NSPLIT)
    def _():
        pltpu.sync_copy(zbuf.at[pl.ds(0, NROWS)],
                        deg_sh.at[pl.ds(s * NROWS, NROWS)])

    plsc.subcore_barrier()
    g0 = w * EPW

    def body(i, carry):
        off = g0 + i * K
        pltpu.sync_copy(dst_hbm.at[pl.ds(off, K)], idx_v)
        pltpu.sync_copy(ew_hbm.at[pl.ds(off, K)], val_v)
        pltpu.sync_copy(val_v, deg_sh.at[idx_v], add=True)
        return carry

    lax.fori_loop(0, ITERS, body, 0)
    plsc.subcore_barrier()

    @pl.when(s < NSPLIT)
    def _():
        pltpu.sync_copy(deg_sh.at[pl.ds(s * NROWS, NROWS)],
                        zbuf.at[pl.ds(0, NROWS)])

    @pl.when(jnp.logical_and(s < NSPLIT, c == 0))
    def _():
        pltpu.sync_copy(zbuf.at[pl.ds(0, NROWS)],
                        pdeg0_hbm.at[pl.ds(s * NROWS, NROWS)])

    @pl.when(jnp.logical_and(s < NSPLIT, c == 1))
    def _():
        pltpu.sync_copy(zbuf.at[pl.ds(0, NROWS)],
                        pdeg1_hbm.at[pl.ds(s * NROWS, NROWS)])


EP = 655360
ECH = EP // K
CPT = ECH // NW
BLKC = 8
_WB = tuple((off, min(K, NROWS - off)) for off in range(0, NROWS, K))


def _scale_rows(rows_ref, ew_ref, r, D):
    def grp(g, cy):
        wv = ew_ref[r, pl.ds(g * 16, 16)]
        base = g * 16
        for jj in range(16):
            wk = wv[jj]
            for ch in range(D // 16):
                sl = pl.ds(ch * 16, 16)
                rows_ref[base + jj, sl] = rows_ref[base + jj, sl] * wk
        return cy

    lax.fori_loop(0, K // 16, grp, 0)


def _agg_body(D, hs_hbm, src_hbm, dst_hbm, ew_hbm, part_hbm,
              sidx2, didx2, ew2, rows0, rows1, gsem0, gsem1, acc_sh):
    c = lax.axis_index("c")
    s = lax.axis_index("s")
    w = c * NS + s
    f32 = jnp.float32

    def zrow(r, cy):
        for ch in range(D // 16):
            rows0[r, pl.ds(ch * 16, 16)] = jnp.zeros((16,), f32)
        return cy

    lax.fori_loop(0, K, zrow, 0)

    @pl.when(s < NSPLIT)
    def _():
        for off, size in _WB:
            pltpu.sync_copy(rows0.at[pl.ds(0, size)],
                            acc_sh.at[pl.ds(s * NROWS + off, size)])

    plsc.subcore_barrier()

    def blk_body(bi, bcarry):
        base = w * CPT + bi * BLKC
        pltpu.sync_copy(src_hbm.at[pl.ds(base, BLKC), :], sidx2)
        pltpu.sync_copy(dst_hbm.at[pl.ds(base, BLKC), :], didx2)
        pltpu.sync_copy(ew_hbm.at[pl.ds(base, BLKC), :], ew2)

        for t in range(BLKC):
            pltpu.async_copy(hs_hbm.at[sidx2.at[t]], rows0, gsem0).wait()
            _scale_rows(rows0, ew2, t, D)
            pltpu.sync_copy(rows0, acc_sh.at[didx2.at[t]], add=True)
        return bcarry

    lax.fori_loop(0, CPT // BLKC, blk_body, 0)
    plsc.subcore_barrier()

    @pl.when(s < NSPLIT)
    def _():
        for off, size in _WB:
            row = s * NROWS + off
            pltpu.sync_copy(acc_sh.at[pl.ds(row, size)],
                            rows0.at[pl.ds(0, size)])
            pltpu.sync_copy(rows0.at[pl.ds(0, size)],
                            part_hbm.at[c, pl.ds(row, size)])


def _make_deg():
    return pl.kernel(
        _deg_body,
        out_type=(jax.ShapeDtypeStruct((N,), jnp.float32),
                  jax.ShapeDtypeStruct((N,), jnp.float32)),
        mesh=_mesh,
        scratch_types=[
            pltpu.VMEM((K,), jnp.int32),
            pltpu.VMEM((K,), jnp.float32),
            pltpu.VMEM((1024,), jnp.float32),
            pltpu.VMEM_SHARED((N,), jnp.float32),
        ],
    )


def _make_agg(D):
    return pl.kernel(
        functools.partial(_agg_body, D),
        out_type=jax.ShapeDtypeStruct((NC, N, D), jnp.float32),
        mesh=_mesh,
        scratch_types=[
            pltpu.VMEM((BLKC, K), jnp.int32),
            pltpu.VMEM((BLKC, K), jnp.int32),
            pltpu.VMEM((BLKC, K), jnp.float32),
            pltpu.VMEM((K, D), jnp.float32),
            pltpu.VMEM((K, D), jnp.float32),
            pltpu.SemaphoreType.DMA,
            pltpu.SemaphoreType.DMA,
            pltpu.VMEM_SHARED((N, D), jnp.float32),
        ],
    )


def _prep_body(pdeg_ref, x_ref, w1_ref, dinv_ref, hs_ref):
    deg = pdeg_ref[:, 0:1] + pdeg_ref[:, 1:2] + 1.0
    dinv = lax.rsqrt(deg)
    dinv_ref[...] = dinv
    h = jnp.dot(x_ref[...], w1_ref[...], preferred_element_type=jnp.float32)
    hs_ref[...] = h * dinv


def _mid_body(p_ref, hs_ref, dinv_ref, b_ref, g_ref, be_ref, w_ref, out_ref):
    dinv = dinv_ref[...]
    agg = p_ref[0] + p_ref[1] + hs_ref[...]
    o = agg * dinv + b_ref[...]
    mean = jnp.mean(o, axis=0, keepdims=True)
    var = jnp.mean(o * o, axis=0, keepdims=True) - mean * mean
    y = (o - mean) * lax.rsqrt(var + 1e-5) * g_ref[...] + be_ref[...]
    y = jnp.maximum(y, 0.0)
    h = jnp.dot(y, w_ref[...], preferred_element_type=jnp.float32)
    out_ref[...] = h * dinv


def _mid_noW_body(p_ref, hs_ref, dinv_ref, b_ref, g_ref, be_ref, out_ref):
    dinv = dinv_ref[...]
    agg = p_ref[0] + p_ref[1] + hs_ref[...]
    o = agg * dinv + b_ref[...]
    mean = jnp.mean(o, axis=0, keepdims=True)
    var = jnp.mean(o * o, axis=0, keepdims=True) - mean * mean
    y = (o - mean) * lax.rsqrt(var + 1e-5) * g_ref[...] + be_ref[...]
    y = jnp.maximum(y, 0.0)
    out_ref[...] = y * dinv


def _final_body(p_ref, q_ref, dinv_ref, w3_ref, b_ref, out_ref):
    agg = (p_ref[0] + p_ref[1] + q_ref[...]) * dinv_ref[...]
    out_ref[...] = (
        jnp.dot(agg, w3_ref[...], preferred_element_type=jnp.float32)
        + b_ref[...])


def kernel(x, edge_index, edge_weight, W1, b1, g1, be1, W2, b2, g2, be2, W3, b3):
    pad = EP - E
    src = jnp.pad(edge_index[0], (0, pad)).reshape(ECH, K)
    dst = jnp.pad(edge_index[1], (0, pad)).reshape(ECH, K)
    eww = jnp.pad(edge_weight, (0, pad)).reshape(ECH, K)
    f32 = jnp.float32

    W3p = jnp.pad(W3, ((0, 0), (0, 16 - W3.shape[1])))
    b3p = jnp.pad(b3, (0, 16 - b3.shape[0]))


    pdeg0, pdeg1 = _make_deg()(edge_index[1], edge_weight)
    pdeg_t = jnp.stack([pdeg0, pdeg1], axis=1)

    dinv, hs1 = pl.pallas_call(
        _prep_body,
        out_shape=(jax.ShapeDtypeStruct((N, 1), f32),
                   jax.ShapeDtypeStruct((N, 128), f32)),
    )(pdeg_t, x, W1)

    agg = _make_agg(128)

    p1 = agg(hs1, src, dst, eww)
    hs2 = pl.pallas_call(
        _mid_body,
        out_shape=jax.ShapeDtypeStruct((N, 128), f32),
    )(p1, hs1, dinv, b1.reshape(1, -1), g1.reshape(1, -1), be1.reshape(1, -1), W2)

    p2 = agg(hs2, src, dst, eww)
    q3 = pl.pallas_call(
        _mid_noW_body,
        out_shape=jax.ShapeDtypeStruct((N, 128), f32),
    )(p2, hs2, dinv, b2.reshape(1, -1), g2.reshape(1, -1), be2.reshape(1, -1))

    p3 = agg(q3, src, dst, eww)
    out = pl.pallas_call(
        _final_body,
        out_shape=jax.ShapeDtypeStruct((N, 16), f32),
    )(p3, q3, dinv, W3p, b3p.reshape(1, -1))
    return out[:, :12]


# speedup vs baseline: 1.8968x; 1.8968x over previous
import functools

import jax
import jax.numpy as jnp
from jax import lax
from jax.experimental import pallas as pl
from jax.experimental.pallas import tpu as pltpu
from jax.experimental.pallas import tpu_sc as plsc

N = 10000
E = 640000
NC = 2
NS = 16
NW = NC * NS
EPW = E // NW
K = 80
ITERS = EPW // K
NROWS = 1000
NSPLIT = N // NROWS

_mesh = plsc.VectorSubcoreMesh(core_axis_name="c", subcore_axis_name="s")


def _deg_body(dst_hbm, ew_hbm, pdeg0_hbm, pdeg1_hbm, idx_v, val_v, zbuf,
              deg_sh):
    c = lax.axis_index("c")
    s = lax.axis_index("s")
    w = c * NS + s

    def zfill(i, cy):
        zbuf[pl.ds(i * 16, 16)] = jnp.zeros((16,), jnp.float32)
        return cy

    lax.fori_loop(0, 64, zfill, 0)

    @pl.when(s < NSPLIT)
    def _():
        pltpu.sync_copy(zbuf.at[pl.ds(0, NROWS)],
                        deg_sh.at[pl.ds(s * NROWS, NROWS)])

    plsc.subcore_barrier()
    g0 = w * EPW

    def body(i, carry):
        off = g0 + i * K
        pltpu.sync_copy(dst_hbm.at[pl.ds(off, K)], idx_v)
        pltpu.sync_copy(ew_hbm.at[pl.ds(off, K)], val_v)
        pltpu.sync_copy(val_v, deg_sh.at[idx_v], add=True)
        return carry

    lax.fori_loop(0, ITERS, body, 0)
    plsc.subcore_barrier()

    @pl.when(s < NSPLIT)
    def _():
        pltpu.sync_copy(deg_sh.at[pl.ds(s * NROWS, NROWS)],
                        zbuf.at[pl.ds(0, NROWS)])

    @pl.when(jnp.logical_and(s < NSPLIT, c == 0))
    def _():
        pltpu.sync_copy(zbuf.at[pl.ds(0, NROWS)],
                        pdeg0_hbm.at[pl.ds(s * NROWS, NROWS)])

    @pl.when(jnp.logical_and(s < NSPLIT, c == 1))
    def _():
        pltpu.sync_copy(zbuf.at[pl.ds(0, NROWS)],
                        pdeg1_hbm.at[pl.ds(s * NROWS, NROWS)])


_WB = tuple((off, min(K, NROWS - off)) for off in range(0, NROWS, K))


def _scale_rows(rows_ref, ew_ref, D):
    def grp(g, cy):
        wv = ew_ref[pl.ds(g * 16, 16)]
        base = g * 16
        for jj in range(16):
            wk = wv[jj]
            for ch in range(D // 16):
                sl = pl.ds(ch * 16, 16)
                rows_ref[base + jj, sl] = rows_ref[base + jj, sl] * wk
        return cy

    lax.fori_loop(0, K // 16, grp, 0)


def _agg_body(D, hs_hbm, src_hbm, dst_hbm, ew_hbm, part_hbm,
              sa, da, wa, sb, db, wb, rows0, rows1, gsem0, gsem1, acc_sh):
    c = lax.axis_index("c")
    s = lax.axis_index("s")
    w = c * NS + s
    f32 = jnp.float32

    def zrow(r, cy):
        for ch in range(D // 16):
            rows0[r, pl.ds(ch * 16, 16)] = jnp.zeros((16,), f32)
        return cy

    lax.fori_loop(0, K, zrow, 0)

    @pl.when(s < NSPLIT)
    def _():
        for off, size in _WB:
            pltpu.sync_copy(rows0.at[pl.ds(0, size)],
                            acc_sh.at[pl.ds(s * NROWS + off, size)])

    plsc.subcore_barrier()
    g0 = w * EPW

    def load_idx(si, di, wi, off):
        pltpu.sync_copy(src_hbm.at[pl.ds(off, K)], si)
        pltpu.sync_copy(dst_hbm.at[pl.ds(off, K)], di)
        pltpu.sync_copy(ew_hbm.at[pl.ds(off, K)], wi)

    load_idx(sa, da, wa, g0)
    pltpu.async_copy(hs_hbm.at[sa], rows0, gsem0)

    def body(j, carry):
        offb = g0 + (2 * j + 1) * K
        load_idx(sb, db, wb, offb)
        pltpu.make_async_copy(hs_hbm.at[sa], rows0, gsem0).wait()
        pltpu.async_copy(hs_hbm.at[sb], rows1, gsem1)
        _scale_rows(rows0, wa, D)
        pltpu.sync_copy(rows0, acc_sh.at[da], add=True)
        pltpu.make_async_copy(hs_hbm.at[sb], rows1, gsem1).wait()
        _scale_rows(rows1, wb, D)

        @pl.when(j < ITERS // 2 - 1)
        def _():
            load_idx(sa, da, wa, offb + K)
            pltpu.async_copy(hs_hbm.at[sa], rows0, gsem0)

        pltpu.sync_copy(rows1, acc_sh.at[db], add=True)
        return carry

    lax.fori_loop(0, ITERS // 2, body, 0)
    plsc.subcore_barrier()

    @pl.when(s < NSPLIT)
    def _():
        for off, size in _WB:
            row = s * NROWS + off
            pltpu.sync_copy(acc_sh.at[pl.ds(row, size)],
                            rows0.at[pl.ds(0, size)])
            pltpu.sync_copy(rows0.at[pl.ds(0, size)],
                            part_hbm.at[c, pl.ds(row, size)])


def _make_deg():
    return pl.kernel(
        _deg_body,
        out_type=(jax.ShapeDtypeStruct((N,), jnp.float32),
                  jax.ShapeDtypeStruct((N,), jnp.float32)),
        mesh=_mesh,
        scratch_types=[
            pltpu.VMEM((K,), jnp.int32),
            pltpu.VMEM((K,), jnp.float32),
            pltpu.VMEM((1024,), jnp.float32),
            pltpu.VMEM_SHARED((N,), jnp.float32),
        ],
    )


def _make_agg(D):
    return pl.kernel(
        functools.partial(_agg_body, D),
        out_type=jax.ShapeDtypeStruct((NC, N, D), jnp.float32),
        mesh=_mesh,
        scratch_types=[
            pltpu.VMEM((K,), jnp.int32),
            pltpu.VMEM((K,), jnp.int32),
            pltpu.VMEM((K,), jnp.float32),
            pltpu.VMEM((K,), jnp.int32),
            pltpu.VMEM((K,), jnp.int32),
            pltpu.VMEM((K,), jnp.float32),
            pltpu.VMEM((K, D), jnp.float32),
            pltpu.VMEM((K, D), jnp.float32),
            pltpu.SemaphoreType.DMA,
            pltpu.SemaphoreType.DMA,
            pltpu.VMEM_SHARED((N, D), jnp.float32),
        ],
    )


def _prep_body(pdeg_ref, x_ref, w1_ref, dinv_ref, hs_ref):
    deg = pdeg_ref[:, 0:1] + pdeg_ref[:, 1:2] + 1.0
    dinv = lax.rsqrt(deg)
    dinv_ref[...] = dinv
    h = jnp.dot(x_ref[...], w1_ref[...], preferred_element_type=jnp.float32)
    hs_ref[...] = h * dinv


def _mid_body(p_ref, hs_ref, dinv_ref, b_ref, g_ref, be_ref, w_ref, out_ref):
    dinv = dinv_ref[...]
    agg = p_ref[0] + p_ref[1] + hs_ref[...]
    o = agg * dinv + b_ref[...]
    mean = jnp.mean(o, axis=0, keepdims=True)
    var = jnp.mean(o * o, axis=0, keepdims=True) - mean * mean
    y = (o - mean) * lax.rsqrt(var + 1e-5) * g_ref[...] + be_ref[...]
    y = jnp.maximum(y, 0.0)
    h = jnp.dot(y, w_ref[...], preferred_element_type=jnp.float32)
    out_ref[...] = h * dinv


def _mid_noW_body(p_ref, hs_ref, dinv_ref, b_ref, g_ref, be_ref, out_ref):
    dinv = dinv_ref[...]
    agg = p_ref[0] + p_ref[1] + hs_ref[...]
    o = agg * dinv + b_ref[...]
    mean = jnp.mean(o, axis=0, keepdims=True)
    var = jnp.mean(o * o, axis=0, keepdims=True) - mean * mean
    y = (o - mean) * lax.rsqrt(var + 1e-5) * g_ref[...] + be_ref[...]
    y = jnp.maximum(y, 0.0)
    out_ref[...] = y * dinv


def _final_body(p_ref, q_ref, dinv_ref, w3_ref, b_ref, out_ref):
    agg = (p_ref[0] + p_ref[1] + q_ref[...]) * dinv_ref[...]
    out_ref[...] = (
        jnp.dot(agg, w3_ref[...], preferred_element_type=jnp.float32)
        + b_ref[...])


def kernel(x, edge_index, edge_weight, W1, b1, g1, be1, W2, b2, g2, be2, W3, b3):
    src = edge_index[0]
    dst = edge_index[1]
    f32 = jnp.float32

    W3p = jnp.pad(W3, ((0, 0), (0, 16 - W3.shape[1])))
    b3p = jnp.pad(b3, (0, 16 - b3.shape[0]))


    pdeg0, pdeg1 = _make_deg()(dst, edge_weight)
    pdeg_t = jnp.stack([pdeg0, pdeg1], axis=1)

    dinv, hs1 = pl.pallas_call(
        _prep_body,
        out_shape=(jax.ShapeDtypeStruct((N, 1), f32),
                   jax.ShapeDtypeStruct((N, 128), f32)),
    )(pdeg_t, x, W1)

    agg = _make_agg(128)

    p1 = agg(hs1, src, dst, edge_weight)
    hs2 = pl.pallas_call(
        _mid_body,
        out_shape=jax.ShapeDtypeStruct((N, 128), f32),
    )(p1, hs1, dinv, b1.reshape(1, -1), g1.reshape(1, -1), be1.reshape(1, -1), W2)

    p2 = agg(hs2, src, dst, edge_weight)
    q3 = pl.pallas_call(
        _mid_noW_body,
        out_shape=jax.ShapeDtypeStruct((N, 128), f32),
    )(p2, hs2, dinv, b2.reshape(1, -1), g2.reshape(1, -1), be2.reshape(1, -1))

    p3 = agg(q3, src, dst, edge_weight)
    out = pl.pallas_call(
        _final_body,
        out_shape=jax.ShapeDtypeStruct((N, 16), f32),
    )(p3, q3, dinv, W3p, b3p.reshape(1, -1))
    return out[:, :12]
